# Initial kernel scaffold; baseline (speedup 1.0000x reference)
import jax
import jax.numpy as jnp
from jax.experimental import pallas as pl


def kernel(x, adj_index, pe_k_table, pe_v_table, basis, comp, root, bias):
    raise NotImplementedError("write your pallas kernel here")


# speedup vs baseline: 30.4978x; 30.4978x over previous
import functools

import jax
import jax.numpy as jnp
from jax import lax
from jax.experimental import pallas as pl
from jax.experimental.pallas import tpu as pltpu
from jax.experimental.pallas import tpu_sc as plsc

WINDOW = 10
UTTER_DIM = 300
NUM_BASES = 4
MAX_LEN = 256
POSI_DIM = 64
REL_NUM = WINDOW + 2
SLEN = 256


def _rgcn_tc_body(x_ref, comp_ref, basis_ref, root_ref, bias_ref, out_ref):
    jj = lax.broadcasted_iota(jnp.int32, (SLEN, SLEN), 0)
    ii = lax.broadcasted_iota(jnp.int32, (SLEN, SLEN), 1)
    d = ii - jj
    dd = jnp.maximum(d, 1)
    m = jnp.minimum((dd + 1) // 2, WINDOW + 1)
    T = jnp.where(ii < jj, 1, jnp.where(ii == jj, 0, REL_NUM - m))
    cnt1 = jj + jnp.maximum(0, (SLEN - 2 * WINDOW - 1) - jj)
    cnt_band = jnp.clip(SLEN + 1 - jj - 2 * m, 0, 2)
    cnt = jnp.where(T == 0, 1, jnp.where(T == 1, cnt1, cnt_band))
    inv_cnt = 1.0 / jnp.maximum(cnt.astype(jnp.float32), 1.0)

    onehot = [(T == t).astype(jnp.float32) for t in range(REL_NUM)]
    for b in range(NUM_BASES):
        numer = jnp.zeros((SLEN, SLEN), jnp.float32)
        for t in range(REL_NUM):
            numer = numer + onehot[t] * comp_ref[t : t + 1, b : b + 1]
        Mb = numer * inv_cnt
        for bt in range(x_ref.shape[0]):
            xi = x_ref[bt]
            mixed = jnp.dot(Mb, xi, preferred_element_type=jnp.float32)
            contrib = jnp.dot(mixed, basis_ref[b],
                              preferred_element_type=jnp.float32)
            if b == 0:
                base_term = jnp.dot(xi, root_ref[...],
                                    preferred_element_type=jnp.float32)
                out_ref[bt] = contrib + base_term + bias_ref[...]
            else:
                out_ref[bt] = out_ref[bt] + contrib


def _rgcn_out(x, comp, basis, root, bias):
    B = x.shape[0]
    return pl.pallas_call(
        _rgcn_tc_body,
        out_shape=jax.ShapeDtypeStruct((B, SLEN, UTTER_DIM), jnp.float32),
    )(x, comp, basis, root, bias.reshape(1, UTTER_DIM))


def _make_sc_relemb(B):
    mesh = plsc.VectorSubcoreMesh(core_axis_name="c", subcore_axis_name="s")
    n_workers = 32
    pairs = B * SLEN
    per_w = pairs // n_workers
    t_per_w = SLEN // (n_workers // B)

    @functools.partial(
        pl.kernel,
        mesh=mesh,
        out_type=[
            jax.ShapeDtypeStruct((B, SLEN, SLEN, POSI_DIM), jnp.float32),
            jax.ShapeDtypeStruct((B, SLEN, SLEN, POSI_DIM), jnp.float32),
        ],
        scratch_types=[
            pltpu.VMEM((2 * SLEN, POSI_DIM), jnp.float32),
            pltpu.VMEM((2 * SLEN, POSI_DIM), jnp.float32),
        ],
    )
    def sc_relemb(padk_hbm, padv_hbm, outk_hbm, outv_hbm, padk_v, padv_v):
        cid = lax.axis_index("c")
        sid = lax.axis_index("s")
        wid = sid * 2 + cid
        pltpu.sync_copy(padk_hbm, padk_v)
        pltpu.sync_copy(padv_hbm, padv_v)
        b = wid // (n_workers // B)
        t0 = (wid % (n_workers // B)) * t_per_w
        for k in range(per_w):
            t = t0 + k
            start = SLEN - 1 - t
            pltpu.sync_copy(padk_v.at[pl.ds(start, SLEN)], outk_hbm.at[b, t])
            pltpu.sync_copy(padv_v.at[pl.ds(start, SLEN)], outv_hbm.at[b, t])

    return sc_relemb


def kernel(x, adj_index, pe_k_table, pe_v_table, basis, comp, root, bias):
    del adj_index
    B = x.shape[0]
    zeros = jnp.zeros((SLEN - 1, POSI_DIM), jnp.float32)
    padk = jnp.concatenate([pe_k_table[::-1], zeros], axis=0)
    padv = jnp.concatenate([pe_v_table[::-1], zeros], axis=0)
    rel_emb_k, rel_emb_v = _make_sc_relemb(B)(padk, padv)
    out = _rgcn_out(x, comp, basis, root, bias)
    return (out, rel_emb_k, rel_emb_v)


# speedup vs baseline: 56.3995x; 1.8493x over previous
import functools

import jax
import jax.numpy as jnp
from jax import lax
from jax.experimental import pallas as pl
from jax.experimental.pallas import tpu as pltpu
from jax.experimental.pallas import tpu_sc as plsc

WINDOW = 10
UTTER_DIM = 300
NUM_BASES = 4
MAX_LEN = 256
POSI_DIM = 64
REL_NUM = WINDOW + 2
SLEN = 256


def _rgcn_tc_body(x_ref, comp_ref, basis_ref, root_ref, bias_ref, out_ref):
    jj = lax.broadcasted_iota(jnp.int32, (SLEN, SLEN), 0)
    ii = lax.broadcasted_iota(jnp.int32, (SLEN, SLEN), 1)
    d = ii - jj
    dd = jnp.maximum(d, 1)
    m = jnp.minimum((dd + 1) // 2, WINDOW + 1)
    T = jnp.where(ii < jj, 1, jnp.where(ii == jj, 0, REL_NUM - m))
    cnt1 = jj + jnp.maximum(0, (SLEN - 2 * WINDOW - 1) - jj)
    cnt_band = jnp.clip(SLEN + 1 - jj - 2 * m, 0, 2)
    cnt = jnp.where(T == 0, 1, jnp.where(T == 1, cnt1, cnt_band))
    inv_cnt = 1.0 / jnp.maximum(cnt.astype(jnp.float32), 1.0)

    onehot = [(T == t).astype(jnp.float32) for t in range(REL_NUM)]
    for b in range(NUM_BASES):
        numer = jnp.zeros((SLEN, SLEN), jnp.float32)
        for t in range(REL_NUM):
            numer = numer + onehot[t] * comp_ref[t : t + 1, b : b + 1]
        Mb = numer * inv_cnt
        for bt in range(x_ref.shape[0]):
            xi = x_ref[bt]
            mixed = jnp.dot(Mb, xi, preferred_element_type=jnp.float32)
            contrib = jnp.dot(mixed, basis_ref[b],
                              preferred_element_type=jnp.float32)
            if b == 0:
                base_term = jnp.dot(xi, root_ref[...],
                                    preferred_element_type=jnp.float32)
                out_ref[bt] = contrib + base_term + bias_ref[...]
            else:
                out_ref[bt] = out_ref[bt] + contrib


def _rgcn_out(x, comp, basis, root, bias):
    B = x.shape[0]
    return pl.pallas_call(
        _rgcn_tc_body,
        out_shape=jax.ShapeDtypeStruct((B, SLEN, UTTER_DIM), jnp.float32),
    )(x, comp, basis, root, bias.reshape(1, UTTER_DIM))


def _make_sc_relemb():
    mesh = plsc.VectorSubcoreMesh(core_axis_name="c", subcore_axis_name="s")
    n_workers = 32
    t_per_w = SLEN // n_workers

    @functools.partial(
        pl.kernel,
        mesh=mesh,
        out_type=[
            jax.ShapeDtypeStruct((SLEN, SLEN, POSI_DIM), jnp.float32),
            jax.ShapeDtypeStruct((SLEN, SLEN, POSI_DIM), jnp.float32),
        ],
        scratch_types=[
            pltpu.VMEM((2 * SLEN, POSI_DIM), jnp.float32),
            pltpu.VMEM((2 * SLEN, POSI_DIM), jnp.float32),
        ],
    )
    def sc_relemb(padk_hbm, padv_hbm, outk_hbm, outv_hbm, padk_v, padv_v):
        cid = lax.axis_index("c")
        sid = lax.axis_index("s")
        wid = sid * 2 + cid
        pltpu.sync_copy(padk_hbm, padk_v)
        pltpu.sync_copy(padv_hbm, padv_v)
        t0 = wid * t_per_w
        for k in range(t_per_w):
            t = t0 + k
            start = SLEN - 1 - t
            pltpu.sync_copy(padk_v.at[pl.ds(start, SLEN)], outk_hbm.at[t])
            pltpu.sync_copy(padv_v.at[pl.ds(start, SLEN)], outv_hbm.at[t])

    return sc_relemb


def kernel(x, adj_index, pe_k_table, pe_v_table, basis, comp, root, bias):
    del adj_index
    B = x.shape[0]
    zeros = jnp.zeros((SLEN - 1, POSI_DIM), jnp.float32)
    padk = jnp.concatenate([pe_k_table[::-1], zeros], axis=0)
    padv = jnp.concatenate([pe_v_table[::-1], zeros], axis=0)
    slab_k, slab_v = _make_sc_relemb()(padk, padv)
    rel_emb_k = jnp.broadcast_to(slab_k[None], (B, SLEN, SLEN, POSI_DIM))
    rel_emb_v = jnp.broadcast_to(slab_v[None], (B, SLEN, SLEN, POSI_DIM))
    out = _rgcn_out(x, comp, basis, root, bias)
    return (out, rel_emb_k, rel_emb_v)


# speedup vs baseline: 57.3011x; 1.0160x over previous
import functools

import jax
import jax.numpy as jnp
from jax import lax
from jax.experimental import pallas as pl
from jax.experimental.pallas import tpu as pltpu
from jax.experimental.pallas import tpu_sc as plsc

WINDOW = 10
UTTER_DIM = 300
NUM_BASES = 4
MAX_LEN = 256
POSI_DIM = 64
REL_NUM = WINDOW + 2
SLEN = 256


def _rgcn_tc_body(x_ref, comp_ref, basis_ref, root_ref, bias_ref, out_ref):
    jj = lax.broadcasted_iota(jnp.int32, (SLEN, SLEN), 0)
    ii = lax.broadcasted_iota(jnp.int32, (SLEN, SLEN), 1)
    d = ii - jj
    dd = jnp.maximum(d, 1)
    m = jnp.minimum((dd + 1) // 2, WINDOW + 1)
    T = jnp.where(ii < jj, 1, jnp.where(ii == jj, 0, REL_NUM - m))
    cnt1 = jj + jnp.maximum(0, (SLEN - 2 * WINDOW - 1) - jj)
    cnt_band = jnp.clip(SLEN + 1 - jj - 2 * m, 0, 2)
    cnt = jnp.where(T == 0, 1, jnp.where(T == 1, cnt1, cnt_band))
    inv_cnt = 1.0 / jnp.maximum(cnt.astype(jnp.float32), 1.0)

    onehot = [(T == t).astype(jnp.float32) for t in range(REL_NUM)]
    for b in range(NUM_BASES):
        numer = jnp.zeros((SLEN, SLEN), jnp.float32)
        for t in range(REL_NUM):
            numer = numer + onehot[t] * comp_ref[t : t + 1, b : b + 1]
        Mb = numer * inv_cnt
        for bt in range(x_ref.shape[0]):
            xi = x_ref[bt]
            mixed = jnp.dot(Mb, xi, preferred_element_type=jnp.float32)
            contrib = jnp.dot(mixed, basis_ref[b],
                              preferred_element_type=jnp.float32)
            if b == 0:
                base_term = jnp.dot(xi, root_ref[...],
                                    preferred_element_type=jnp.float32)
                out_ref[bt] = contrib + base_term + bias_ref[...]
            else:
                out_ref[bt] = out_ref[bt] + contrib


def _rgcn_out(x, comp, basis, root, bias):
    B = x.shape[0]
    return pl.pallas_call(
        _rgcn_tc_body,
        out_shape=jax.ShapeDtypeStruct((B, SLEN, UTTER_DIM), jnp.float32),
    )(x, comp, basis, root, bias.reshape(1, UTTER_DIM))


def _make_sc_relemb():
    mesh = plsc.VectorSubcoreMesh(core_axis_name="c", subcore_axis_name="s")
    n_workers = 32
    t_per_w = SLEN // n_workers

    @functools.partial(
        pl.kernel,
        mesh=mesh,
        out_type=jax.ShapeDtypeStruct((SLEN, SLEN, POSI_DIM), jnp.float32),
        scratch_types=[
            pltpu.VMEM((2 * SLEN, POSI_DIM), jnp.float32),
            pltpu.SemaphoreType.DMA,
        ],
    )
    def sc_relemb(pad_hbm, out_hbm, pad_v, sem):
        cid = lax.axis_index("c")
        sid = lax.axis_index("s")
        wid = sid * 2 + cid
        pltpu.sync_copy(pad_hbm, pad_v)
        t0 = wid * t_per_w
        copies = []
        for k in range(t_per_w):
            t = t0 + k
            start = SLEN - 1 - t
            copies.append(pltpu.make_async_copy(
                pad_v.at[pl.ds(start, SLEN)], out_hbm.at[t], sem))
        for c in copies:
            c.start()
        for c in copies:
            c.wait()

    return sc_relemb


def kernel(x, adj_index, pe_k_table, pe_v_table, basis, comp, root, bias):
    del adj_index
    B = x.shape[0]
    zeros = jnp.zeros((SLEN - 1, POSI_DIM), jnp.float32)
    padk = jnp.concatenate([pe_k_table[::-1], zeros], axis=0)
    padv = jnp.concatenate([pe_v_table[::-1], zeros], axis=0)
    sc_call = _make_sc_relemb()
    slab_k = sc_call(padk)
    slab_v = sc_call(padv)
    rel_emb_k = jnp.broadcast_to(slab_k[None], (B, SLEN, SLEN, POSI_DIM))
    rel_emb_v = jnp.broadcast_to(slab_v[None], (B, SLEN, SLEN, POSI_DIM))
    out = _rgcn_out(x, comp, basis, root, bias)
    return (out, rel_emb_k, rel_emb_v)


# speedup vs baseline: 62.3393x; 1.0879x over previous
import functools

import jax
import jax.numpy as jnp
from jax import lax
from jax.experimental import pallas as pl
from jax.experimental.pallas import tpu as pltpu
from jax.experimental.pallas import tpu_sc as plsc

WINDOW = 10
UTTER_DIM = 300
NUM_BASES = 4
MAX_LEN = 256
POSI_DIM = 64
REL_NUM = WINDOW + 2
SLEN = 256


def _rgcn_tc_body(x_ref, comp_ref, basis_ref, root_ref, bias_ref, out_ref):
    jj = lax.broadcasted_iota(jnp.int32, (SLEN, SLEN), 0)
    ii = lax.broadcasted_iota(jnp.int32, (SLEN, SLEN), 1)
    d = ii - jj
    dd = jnp.maximum(d, 1)
    m = jnp.minimum((dd + 1) // 2, WINDOW + 1)
    T = jnp.where(ii < jj, 1, jnp.where(ii == jj, 0, REL_NUM - m))
    cnt1 = jj + jnp.maximum(0, (SLEN - 2 * WINDOW - 1) - jj)
    cnt_band = jnp.clip(SLEN + 1 - jj - 2 * m, 0, 2)
    cnt = jnp.where(T == 0, 1, jnp.where(T == 1, cnt1, cnt_band))
    inv_cnt = 1.0 / jnp.maximum(cnt.astype(jnp.float32), 1.0)

    onehot = [(T == t).astype(jnp.float32) for t in range(REL_NUM)]
    for b in range(NUM_BASES):
        numer = jnp.zeros((SLEN, SLEN), jnp.float32)
        for t in range(REL_NUM):
            numer = numer + onehot[t] * comp_ref[t : t + 1, b : b + 1]
        Mb = numer * inv_cnt
        for bt in range(x_ref.shape[0]):
            xi = x_ref[bt]
            mixed = jnp.dot(Mb, xi, preferred_element_type=jnp.float32)
            contrib = jnp.dot(mixed, basis_ref[b],
                              preferred_element_type=jnp.float32)
            if b == 0:
                base_term = jnp.dot(xi, root_ref[...],
                                    preferred_element_type=jnp.float32)
                out_ref[bt] = contrib + base_term + bias_ref[...]
            else:
                out_ref[bt] = out_ref[bt] + contrib


def _rgcn_out(x, comp, basis, root, bias):
    B = x.shape[0]
    return pl.pallas_call(
        _rgcn_tc_body,
        out_shape=jax.ShapeDtypeStruct((B, SLEN, UTTER_DIM), jnp.float32),
    )(x, comp, basis, root, bias.reshape(1, UTTER_DIM))


def _make_sc_relemb():
    mesh = plsc.VectorSubcoreMesh(core_axis_name="c", subcore_axis_name="s")
    n_workers = 32
    t_per_w = SLEN // n_workers

    @functools.partial(
        pl.kernel,
        mesh=mesh,
        out_type=jax.ShapeDtypeStruct((SLEN, SLEN, POSI_DIM), jnp.float32),
        scratch_types=[
            pltpu.VMEM((2 * SLEN, POSI_DIM), jnp.float32),
            pltpu.SemaphoreType.DMA,
        ],
    )
    def sc_relemb(pad_hbm, out_hbm, pad_v, sem):
        cid = lax.axis_index("c")
        sid = lax.axis_index("s")
        wid = sid * 2 + cid
        pltpu.sync_copy(pad_hbm, pad_v)
        t0 = wid * t_per_w
        copies = []
        for k in range(t_per_w):
            t = t0 + k
            start = SLEN - 1 - t
            copies.append(pltpu.make_async_copy(
                pad_v.at[pl.ds(start, SLEN)], out_hbm.at[t], sem))
        for c in copies:
            c.start()
        for c in copies:
            c.wait()

    return sc_relemb


_T_BLK = 8
_PT = POSI_DIM // 8
_ST = SLEN // 128


def _assemble_body(slab_ref, out_ref):
    for tl in range(_T_BLK):
        w = slab_ref[tl]
        wt = w.T
        for s0 in range(_ST):
            chunk = wt[:, 128 * s0:128 * (s0 + 1)]
            tile = chunk.reshape(_PT, 8, 128)
            for b in range(4):
                out_ref[b, tl, :, s0] = tile


def _assemble(slab, B):
    grid = SLEN // _T_BLK
    return pl.pallas_call(
        _assemble_body,
        grid=(grid,),
        in_specs=[pl.BlockSpec((_T_BLK, SLEN, POSI_DIM),
                               lambda i: (i, 0, 0))],
        out_specs=pl.BlockSpec((B, _T_BLK, _PT, _ST, 8, 128),
                               lambda i: (0, i, 0, 0, 0, 0)),
        out_shape=jax.ShapeDtypeStruct((B, SLEN, _PT, _ST, 8, 128),
                                       jnp.float32),
    )(slab)


def _untile(out6):
    b, t = out6.shape[:2]
    x = out6.transpose(0, 1, 2, 4, 3, 5).reshape(b, t, POSI_DIM, SLEN)
    return x.transpose(0, 1, 3, 2)


def kernel(x, adj_index, pe_k_table, pe_v_table, basis, comp, root, bias):
    del adj_index
    B = x.shape[0]
    zeros = jnp.zeros((SLEN - 1, POSI_DIM), jnp.float32)
    padk = jnp.concatenate([pe_k_table[::-1], zeros], axis=0)
    padv = jnp.concatenate([pe_v_table[::-1], zeros], axis=0)
    sc_call = _make_sc_relemb()
    slab_k = sc_call(padk)
    rel_emb_k = _untile(_assemble(slab_k, B))
    slab_v = sc_call(padv)
    rel_emb_v = _untile(_assemble(slab_v, B))
    out = _rgcn_out(x, comp, basis, root, bias)
    return (out, rel_emb_k, rel_emb_v)


# speedup vs baseline: 67.5741x; 1.0840x over previous
import functools

import jax
import jax.numpy as jnp
from jax import lax
from jax.experimental import pallas as pl
from jax.experimental.pallas import tpu as pltpu
from jax.experimental.pallas import tpu_sc as plsc

WINDOW = 10
UTTER_DIM = 300
NUM_BASES = 4
MAX_LEN = 256
POSI_DIM = 64
REL_NUM = WINDOW + 2
SLEN = 256


def _rgcn_tc_body(x_ref, comp_ref, basis_ref, root_ref, bias_ref, out_ref):
    jj = lax.broadcasted_iota(jnp.int32, (SLEN, SLEN), 0)
    ii = lax.broadcasted_iota(jnp.int32, (SLEN, SLEN), 1)
    d = ii - jj
    dd = jnp.maximum(d, 1)
    m = jnp.minimum((dd + 1) // 2, WINDOW + 1)
    T = jnp.where(ii < jj, 1, jnp.where(ii == jj, 0, REL_NUM - m))
    cnt1 = jj + jnp.maximum(0, (SLEN - 2 * WINDOW - 1) - jj)
    cnt_band = jnp.clip(SLEN + 1 - jj - 2 * m, 0, 2)
    cnt = jnp.where(T == 0, 1, jnp.where(T == 1, cnt1, cnt_band))
    inv_cnt = 1.0 / jnp.maximum(cnt.astype(jnp.float32), 1.0)

    onehot = [(T == t).astype(jnp.float32) for t in range(REL_NUM)]
    for b in range(NUM_BASES):
        numer = jnp.zeros((SLEN, SLEN), jnp.float32)
        for t in range(REL_NUM):
            numer = numer + onehot[t] * comp_ref[t : t + 1, b : b + 1]
        Mb = numer * inv_cnt
        for bt in range(x_ref.shape[0]):
            xi = x_ref[bt]
            mixed = jnp.dot(Mb, xi, preferred_element_type=jnp.float32)
            contrib = jnp.dot(mixed, basis_ref[b],
                              preferred_element_type=jnp.float32)
            if b == 0:
                base_term = jnp.dot(xi, root_ref[...],
                                    preferred_element_type=jnp.float32)
                out_ref[bt] = contrib + base_term + bias_ref[...]
            else:
                out_ref[bt] = out_ref[bt] + contrib


def _rgcn_out(x, comp, basis, root, bias):
    B = x.shape[0]
    return pl.pallas_call(
        _rgcn_tc_body,
        out_shape=jax.ShapeDtypeStruct((B, SLEN, UTTER_DIM), jnp.float32),
    )(x, comp, basis, root, bias.reshape(1, UTTER_DIM))


def _make_sc_relemb():
    mesh = plsc.VectorSubcoreMesh(core_axis_name="c", subcore_axis_name="s")
    n_workers = 32
    t_per_w = SLEN // n_workers

    @functools.partial(
        pl.kernel,
        mesh=mesh,
        out_type=jax.ShapeDtypeStruct((SLEN, SLEN, POSI_DIM), jnp.float32),
        scratch_types=[
            pltpu.VMEM((2 * SLEN, POSI_DIM), jnp.float32),
            pltpu.SemaphoreType.DMA,
        ],
    )
    def sc_relemb(pad_hbm, out_hbm, pad_v, sem):
        cid = lax.axis_index("c")
        sid = lax.axis_index("s")
        wid = sid * 2 + cid
        pltpu.sync_copy(pad_hbm, pad_v)
        t0 = wid * t_per_w
        copies = []
        for k in range(t_per_w):
            t = t0 + k
            start = SLEN - 1 - t
            copies.append(pltpu.make_async_copy(
                pad_v.at[pl.ds(start, SLEN)], out_hbm.at[t], sem))
        for c in copies:
            c.start()
        for c in copies:
            c.wait()

    return sc_relemb


_T_BLK = 16
_PT = POSI_DIM // 8
_ST = SLEN // 128


def _assemble_body(slab_ref, out_ref):
    for tl in range(_T_BLK):
        w = slab_ref[tl]
        wt = w.T
        for s0 in range(_ST):
            chunk = wt[:, 128 * s0:128 * (s0 + 1)]
            tile = chunk.reshape(_PT, 8, 128)
            for b in range(4):
                out_ref[b, tl, :, s0] = tile


def _assemble(slab, B):
    grid = SLEN // _T_BLK
    return pl.pallas_call(
        _assemble_body,
        grid=(grid,),
        in_specs=[pl.BlockSpec((_T_BLK, SLEN, POSI_DIM),
                               lambda i: (i, 0, 0))],
        out_specs=pl.BlockSpec((B, _T_BLK, _PT, _ST, 8, 128),
                               lambda i: (0, i, 0, 0, 0, 0)),
        out_shape=jax.ShapeDtypeStruct((B, SLEN, _PT, _ST, 8, 128),
                                       jnp.float32),
    )(slab)


def _untile(out6):
    b, t = out6.shape[:2]
    x = out6.transpose(0, 1, 2, 4, 3, 5).reshape(b, t, POSI_DIM, SLEN)
    return x.transpose(0, 1, 3, 2)


def kernel(x, adj_index, pe_k_table, pe_v_table, basis, comp, root, bias):
    del adj_index
    B = x.shape[0]
    zeros = jnp.zeros((SLEN - 1, POSI_DIM), jnp.float32)
    padk = jnp.concatenate([pe_k_table[::-1], zeros], axis=0)
    padv = jnp.concatenate([pe_v_table[::-1], zeros], axis=0)
    sc_call = _make_sc_relemb()
    slab_k = sc_call(padk)
    rel_emb_k = _untile(_assemble(slab_k, B))
    slab_v = sc_call(padv)
    rel_emb_v = _untile(_assemble(slab_v, B))
    out = _rgcn_out(x, comp, basis, root, bias)
    return (out, rel_emb_k, rel_emb_v)


# speedup vs baseline: 71.2527x; 1.0544x over previous
import functools

import jax
import jax.numpy as jnp
from jax import lax
from jax.experimental import pallas as pl
from jax.experimental.pallas import tpu as pltpu
from jax.experimental.pallas import tpu_sc as plsc

WINDOW = 10
UTTER_DIM = 300
NUM_BASES = 4
MAX_LEN = 256
POSI_DIM = 64
REL_NUM = WINDOW + 2
SLEN = 256


def _rgcn_tc_body(x_ref, comp_ref, basis_ref, root_ref, bias_ref, out_ref):
    jj = lax.broadcasted_iota(jnp.int32, (SLEN, SLEN), 0)
    ii = lax.broadcasted_iota(jnp.int32, (SLEN, SLEN), 1)
    d = ii - jj
    dd = jnp.maximum(d, 1)
    m = jnp.minimum((dd + 1) // 2, WINDOW + 1)
    T = jnp.where(ii < jj, 1, jnp.where(ii == jj, 0, REL_NUM - m))
    cnt1 = jj + jnp.maximum(0, (SLEN - 2 * WINDOW - 1) - jj)
    cnt_band = jnp.clip(SLEN + 1 - jj - 2 * m, 0, 2)
    cnt = jnp.where(T == 0, 1, jnp.where(T == 1, cnt1, cnt_band))
    inv_cnt = 1.0 / jnp.maximum(cnt.astype(jnp.float32), 1.0)

    onehot = [(T == t).astype(jnp.float32) for t in range(REL_NUM)]
    for b in range(NUM_BASES):
        numer = jnp.zeros((SLEN, SLEN), jnp.float32)
        for t in range(REL_NUM):
            numer = numer + onehot[t] * comp_ref[t : t + 1, b : b + 1]
        Mb = numer * inv_cnt
        for bt in range(x_ref.shape[0]):
            xi = x_ref[bt]
            mixed = jnp.dot(Mb, xi, preferred_element_type=jnp.float32)
            contrib = jnp.dot(mixed, basis_ref[b],
                              preferred_element_type=jnp.float32)
            if b == 0:
                base_term = jnp.dot(xi, root_ref[...],
                                    preferred_element_type=jnp.float32)
                out_ref[bt] = contrib + base_term + bias_ref[...]
            else:
                out_ref[bt] = out_ref[bt] + contrib


def _rgcn_out(x, comp, basis, root, bias):
    B = x.shape[0]
    return pl.pallas_call(
        _rgcn_tc_body,
        out_shape=jax.ShapeDtypeStruct((B, SLEN, UTTER_DIM), jnp.float32),
    )(x, comp, basis, root, bias.reshape(1, UTTER_DIM))


def _make_sc_relemb():
    mesh = plsc.VectorSubcoreMesh(core_axis_name="c", subcore_axis_name="s")
    n_workers = 32
    t_per_w = SLEN // n_workers

    @functools.partial(
        pl.kernel,
        mesh=mesh,
        out_type=jax.ShapeDtypeStruct((SLEN, SLEN, POSI_DIM), jnp.float32),
        scratch_types=[
            pltpu.VMEM((2 * SLEN, POSI_DIM), jnp.float32),
            pltpu.SemaphoreType.DMA,
        ],
    )
    def sc_relemb(pad_hbm, out_hbm, pad_v, sem):
        cid = lax.axis_index("c")
        sid = lax.axis_index("s")
        wid = sid * 2 + cid
        pltpu.sync_copy(pad_hbm, pad_v)
        t0 = wid * t_per_w
        copies = []
        for k in range(t_per_w):
            t = t0 + k
            start = SLEN - 1 - t
            copies.append(pltpu.make_async_copy(
                pad_v.at[pl.ds(start, SLEN)], out_hbm.at[t], sem))
        for c in copies:
            c.start()
        for c in copies:
            c.wait()

    return sc_relemb


_T_BLK = 32
_PT = POSI_DIM // 8
_ST = SLEN // 128


def _assemble_body(slab_ref, out_ref):
    for tl in range(_T_BLK):
        w = slab_ref[tl]
        wt = w.T
        for s0 in range(_ST):
            chunk = wt[:, 128 * s0:128 * (s0 + 1)]
            tile = chunk.reshape(_PT, 8, 128)
            for b in range(4):
                out_ref[b, tl, :, s0] = tile


def _assemble(slab, B):
    grid = SLEN // _T_BLK
    return pl.pallas_call(
        _assemble_body,
        grid=(grid,),
        in_specs=[pl.BlockSpec((_T_BLK, SLEN, POSI_DIM),
                               lambda i: (i, 0, 0))],
        out_specs=pl.BlockSpec((B, _T_BLK, _PT, _ST, 8, 128),
                               lambda i: (0, i, 0, 0, 0, 0)),
        out_shape=jax.ShapeDtypeStruct((B, SLEN, _PT, _ST, 8, 128),
                                       jnp.float32),
    )(slab)


def _untile(out6):
    b, t = out6.shape[:2]
    x = out6.transpose(0, 1, 2, 4, 3, 5).reshape(b, t, POSI_DIM, SLEN)
    return x.transpose(0, 1, 3, 2)


def kernel(x, adj_index, pe_k_table, pe_v_table, basis, comp, root, bias):
    del adj_index
    B = x.shape[0]
    zeros = jnp.zeros((SLEN - 1, POSI_DIM), jnp.float32)
    padk = jnp.concatenate([pe_k_table[::-1], zeros], axis=0)
    padv = jnp.concatenate([pe_v_table[::-1], zeros], axis=0)
    sc_call = _make_sc_relemb()
    slab_k = sc_call(padk)
    rel_emb_k = _untile(_assemble(slab_k, B))
    slab_v = sc_call(padv)
    rel_emb_v = _untile(_assemble(slab_v, B))
    out = _rgcn_out(x, comp, basis, root, bias)
    return (out, rel_emb_k, rel_emb_v)


# speedup vs baseline: 72.2703x; 1.0143x over previous
import functools

import jax
import jax.numpy as jnp
from jax import lax
from jax.experimental import pallas as pl
from jax.experimental.pallas import tpu as pltpu
from jax.experimental.pallas import tpu_sc as plsc

WINDOW = 10
UTTER_DIM = 300
NUM_BASES = 4
MAX_LEN = 256
POSI_DIM = 64
REL_NUM = WINDOW + 2
SLEN = 256


def _rgcn_tc_body(x_ref, comp_ref, basis_ref, root_ref, bias_ref, out_ref):
    jj = lax.broadcasted_iota(jnp.int32, (SLEN, SLEN), 0)
    ii = lax.broadcasted_iota(jnp.int32, (SLEN, SLEN), 1)
    d = ii - jj
    dd = jnp.maximum(d, 1)
    m = jnp.minimum((dd + 1) // 2, WINDOW + 1)
    T = jnp.where(ii < jj, 1, jnp.where(ii == jj, 0, REL_NUM - m))
    cnt1 = jj + jnp.maximum(0, (SLEN - 2 * WINDOW - 1) - jj)
    cnt_band = jnp.clip(SLEN + 1 - jj - 2 * m, 0, 2)
    cnt = jnp.where(T == 0, 1, jnp.where(T == 1, cnt1, cnt_band))
    inv_cnt = 1.0 / jnp.maximum(cnt.astype(jnp.float32), 1.0)

    onehot = [(T == t).astype(jnp.float32) for t in range(REL_NUM)]
    for b in range(NUM_BASES):
        numer = jnp.zeros((SLEN, SLEN), jnp.float32)
        for t in range(REL_NUM):
            numer = numer + onehot[t] * comp_ref[t : t + 1, b : b + 1]
        Mb = numer * inv_cnt
        for bt in range(x_ref.shape[0]):
            xi = x_ref[bt]
            mixed = jnp.dot(Mb, xi, preferred_element_type=jnp.float32)
            contrib = jnp.dot(mixed, basis_ref[b],
                              preferred_element_type=jnp.float32)
            if b == 0:
                base_term = jnp.dot(xi, root_ref[...],
                                    preferred_element_type=jnp.float32)
                out_ref[bt] = contrib + base_term + bias_ref[...]
            else:
                out_ref[bt] = out_ref[bt] + contrib


def _rgcn_out(x, comp, basis, root, bias):
    B = x.shape[0]
    return pl.pallas_call(
        _rgcn_tc_body,
        out_shape=jax.ShapeDtypeStruct((B, SLEN, UTTER_DIM), jnp.float32),
    )(x, comp, basis, root, bias.reshape(1, UTTER_DIM))


def _make_sc_relemb():
    mesh = plsc.VectorSubcoreMesh(core_axis_name="c", subcore_axis_name="s")
    n_workers = 32
    t_per_w = SLEN // n_workers

    @functools.partial(
        pl.kernel,
        mesh=mesh,
        out_type=jax.ShapeDtypeStruct((SLEN, SLEN, POSI_DIM), jnp.float32),
        scratch_types=[
            pltpu.VMEM((2 * SLEN, POSI_DIM), jnp.float32),
            pltpu.SemaphoreType.DMA,
        ],
    )
    def sc_relemb(pad_hbm, out_hbm, pad_v, sem):
        cid = lax.axis_index("c")
        sid = lax.axis_index("s")
        wid = sid * 2 + cid
        pltpu.sync_copy(pad_hbm, pad_v)
        t0 = wid * t_per_w
        copies = []
        for k in range(t_per_w):
            t = t0 + k
            start = SLEN - 1 - t
            copies.append(pltpu.make_async_copy(
                pad_v.at[pl.ds(start, SLEN)], out_hbm.at[t], sem))
        for c in copies:
            c.start()
        for c in copies:
            c.wait()

    return sc_relemb


_T_BLK = 64
_PT = POSI_DIM // 8
_ST = SLEN // 128


def _assemble_body(slab_ref, out_ref):
    for tl in range(_T_BLK):
        w = slab_ref[tl]
        wt = w.T
        for s0 in range(_ST):
            chunk = wt[:, 128 * s0:128 * (s0 + 1)]
            tile = chunk.reshape(_PT, 8, 128)
            for b in range(4):
                out_ref[b, tl, :, s0] = tile


def _assemble(slab, B):
    grid = SLEN // _T_BLK
    return pl.pallas_call(
        _assemble_body,
        grid=(grid,),
        in_specs=[pl.BlockSpec((_T_BLK, SLEN, POSI_DIM),
                               lambda i: (i, 0, 0))],
        out_specs=pl.BlockSpec((B, _T_BLK, _PT, _ST, 8, 128),
                               lambda i: (0, i, 0, 0, 0, 0)),
        out_shape=jax.ShapeDtypeStruct((B, SLEN, _PT, _ST, 8, 128),
                                       jnp.float32),
    )(slab)


def _untile(out6):
    b, t = out6.shape[:2]
    x = out6.transpose(0, 1, 2, 4, 3, 5).reshape(b, t, POSI_DIM, SLEN)
    return x.transpose(0, 1, 3, 2)


def kernel(x, adj_index, pe_k_table, pe_v_table, basis, comp, root, bias):
    del adj_index
    B = x.shape[0]
    zeros = jnp.zeros((SLEN - 1, POSI_DIM), jnp.float32)
    padk = jnp.concatenate([pe_k_table[::-1], zeros], axis=0)
    padv = jnp.concatenate([pe_v_table[::-1], zeros], axis=0)
    sc_call = _make_sc_relemb()
    slab_k = sc_call(padk)
    rel_emb_k = _untile(_assemble(slab_k, B))
    slab_v = sc_call(padv)
    rel_emb_v = _untile(_assemble(slab_v, B))
    out = _rgcn_out(x, comp, basis, root, bias)
    return (out, rel_emb_k, rel_emb_v)
